# baseline (speedup 1.0000x reference)
import jax
import jax.numpy as jnp
from jax import lax
from jax.experimental import pallas as pl
from jax.experimental.pallas import tpu as pltpu

N_EXPERT = 8
TOP_K = 2
N_EMBD = 2048
D_FF = 4096

NEG_BIG = -1e30


def _gate_body(x_ref, wg_ref, o_ref):
    s = lax.dot_general(
        x_ref[...], wg_ref[...],
        dimension_numbers=(((1,), (1,)), ((), ())),
        preferred_element_type=jnp.float32,
    )
    n, e = s.shape
    col = lax.broadcasted_iota(jnp.int32, (n, e), 1)
    m1 = jnp.max(s, axis=1, keepdims=True)
    i1 = jnp.min(jnp.where(s == m1, col, e), axis=1, keepdims=True)
    s2 = jnp.where(col == i1, NEG_BIG, s)
    m2 = jnp.max(s2, axis=1, keepdims=True)
    i2 = jnp.min(jnp.where(s2 == m2, col, e), axis=1, keepdims=True)
    t = jnp.exp(m2 - m1)
    p1 = 1.0 / (1.0 + t)
    p2 = 1.0 - p1
    o_ref[...] = jnp.where(col == i1, p1, 0.0) + jnp.where(col == i2, p2, 0.0)


def _gate(x_flat, Wg):
    n = x_flat.shape[0]
    return pl.pallas_call(
        _gate_body,
        out_shape=jax.ShapeDtypeStruct((n, N_EXPERT), jnp.float32),
    )(x_flat, Wg)


RB = 256
FB = 512


def _mlp_body(w8_ref, x_ref, w1_ref, w2_ref, o_ref):
    e = pl.program_id(0)
    r = pl.program_id(2)

    @pl.when((e == 0) & (pl.program_id(1) == 0))
    def _():
        o_ref[pl.ds(r * RB, RB), :] = jnp.zeros((RB, N_EMBD), jnp.float32)

    xb = x_ref[...].astype(jnp.bfloat16)
    w1b = w1_ref[0].astype(jnp.bfloat16)
    h = lax.dot_general(
        xb, w1b,
        dimension_numbers=(((1,), (1,)), ((), ())),
        preferred_element_type=jnp.float32,
    )
    h = h * (1.0 / (1.0 + jnp.exp(-h)))
    w2b = w2_ref[0].astype(jnp.bfloat16)
    acc = lax.dot_general(
        h.astype(jnp.bfloat16), w2b,
        dimension_numbers=(((1,), (1,)), ((), ())),
        preferred_element_type=jnp.float32,
    )
    col = lax.broadcasted_iota(jnp.int32, (RB, N_EXPERT), 1)
    w = jnp.sum(jnp.where(col == e, w8_ref[...], 0.0), axis=1, keepdims=True)
    o_ref[pl.ds(r * RB, RB), :] += w * acc


def _mlp(x_flat, W1, W2, w8):
    n = x_flat.shape[0]
    grid = (N_EXPERT, D_FF // FB, n // RB)
    return pl.pallas_call(
        _mlp_body,
        grid=grid,
        in_specs=[
            pl.BlockSpec((RB, N_EXPERT), lambda e, f, r: (r, 0)),
            pl.BlockSpec((RB, N_EMBD), lambda e, f, r: (r, 0)),
            pl.BlockSpec((1, FB, N_EMBD), lambda e, f, r: (e, f, 0)),
            pl.BlockSpec((1, N_EMBD, FB), lambda e, f, r: (e, 0, f)),
        ],
        out_specs=pl.BlockSpec((n, N_EMBD), lambda e, f, r: (0, 0)),
        out_shape=jax.ShapeDtypeStruct((n, N_EMBD), jnp.float32),
        compiler_params=pltpu.CompilerParams(
            dimension_semantics=("arbitrary", "arbitrary", "arbitrary"),
        ),
    )(w8, x_flat, W1, W2)


@jax.jit
def kernel(x, Wg, W1, W2):
    B, T, C = x.shape
    x_flat = x.reshape(T, C)
    w8 = _gate(x_flat, Wg)
    y = _mlp(x_flat, W1, W2, w8)
    return y.reshape(B, T, C)


# speedup vs baseline: 1.0998x; 1.0998x over previous
import functools

import jax
import jax.numpy as jnp
from jax import lax
from jax.experimental import pallas as pl
from jax.experimental.pallas import tpu as pltpu
from jax.experimental.pallas import tpu_sc as plsc

N_EXPERT = 8
TOP_K = 2
N_EMBD = 2048
D_FF = 4096
N_TOK = 2048

RB = 128
NB = N_TOK * TOP_K // RB + N_EXPERT
NP = NB * RB
FB = 512
NF = D_FF // FB

NW = 32

NEG_BIG = -1e30


def _gate_body(x_ref, wg_ref, idx_ref, prb_ref):
    s = lax.dot_general(
        x_ref[...], wg_ref[...],
        dimension_numbers=(((1,), (1,)), ((), ())),
        preferred_element_type=jnp.float32,
    )
    n, e = s.shape
    col = lax.broadcasted_iota(jnp.int32, (n, e), 1)
    m1 = jnp.max(s, axis=1, keepdims=True)
    i1 = jnp.min(jnp.where(s == m1, col, e), axis=1, keepdims=True)
    s2 = jnp.where(col == i1, NEG_BIG, s)
    m2 = jnp.max(s2, axis=1, keepdims=True)
    i2 = jnp.min(jnp.where(s2 == m2, col, e), axis=1, keepdims=True)
    t = jnp.exp(m2 - m1)
    p1 = 1.0 / (1.0 + t)
    p2 = 1.0 - p1
    idx_ref[...] = jnp.where(col == 0, i1, jnp.where(col == 1, i2, 0))
    prb_ref[...] = jnp.where(col == 0, p1, jnp.where(col == 1, p2, 0.0))


def _gate(x_flat, Wg):
    n = x_flat.shape[0]
    return pl.pallas_call(
        _gate_body,
        out_shape=(
            jax.ShapeDtypeStruct((n, N_EXPERT), jnp.int32),
            jax.ShapeDtypeStruct((n, N_EXPERT), jnp.float32),
        ),
    )(x_flat, Wg)


def _routing_metadata(idx8, prb8):
    n = idx8.shape[0]
    pair_e = jnp.concatenate([idx8[:, 0], idx8[:, 1]])
    pair_t = jnp.concatenate([jnp.arange(n, dtype=jnp.int32)] * 2)
    pair_p = jnp.concatenate([prb8[:, 0], prb8[:, 1]])

    onehot = (pair_e[:, None] == jnp.arange(N_EXPERT, dtype=jnp.int32)[None, :])
    onehot = onehot.astype(jnp.int32)
    excl = jnp.cumsum(onehot, axis=0) - onehot
    rank = jnp.take_along_axis(excl, pair_e[:, None], axis=1)[:, 0]
    counts = jnp.sum(onehot, axis=0)
    padded = ((counts + RB - 1) // RB) * RB
    offs = jnp.concatenate([jnp.zeros((1,), jnp.int32),
                            jnp.cumsum(padded)[:-1]])
    ppos = offs[pair_e] + rank

    src = jnp.zeros((NP,), jnp.int32).at[ppos].set(pair_t)
    prob = jnp.zeros((NP,), jnp.float32).at[ppos].set(pair_p)
    blk_off = offs // RB
    blk_e = (jnp.searchsorted(blk_off, jnp.arange(NB, dtype=jnp.int32),
                              side="right") - 1).astype(jnp.int32)
    posA = ppos[:n].astype(jnp.int32)
    posB = ppos[n:].astype(jnp.int32)
    return src, prob, blk_e, posA, posB


def _sc_dispatch(idx, table):
    per_w = NP // NW
    ch = 16
    mesh = plsc.VectorSubcoreMesh(core_axis_name="c", subcore_axis_name="s")

    @functools.partial(
        pl.kernel,
        out_type=jax.ShapeDtypeStruct((NP, N_EMBD), jnp.float32),
        mesh=mesh,
        scratch_types=[
            pltpu.VMEM((per_w,), jnp.int32),
            pltpu.VMEM((ch, N_EMBD), jnp.float32),
            pltpu.SemaphoreType.DMA,
        ],
    )
    def k(idx_hbm, table_hbm, out_hbm, idx_v, rows_v, sem):
        wid = lax.axis_index("s") * 2 + lax.axis_index("c")
        base = wid * per_w
        pltpu.sync_copy(idx_hbm.at[pl.ds(base, per_w)], idx_v)

        def body(c, carry):
            pltpu.async_copy(
                table_hbm.at[idx_v.at[pl.ds(c * ch, ch)]], rows_v, sem
            ).wait()
            pltpu.sync_copy(rows_v, out_hbm.at[pl.ds(base + c * ch, ch)])
            return carry

        lax.fori_loop(0, per_w // ch, body, 0)

    return k(idx, table)


def _sc_combine(posA, posB, table):
    per_w = N_TOK // NW
    ch = 16
    mesh = plsc.VectorSubcoreMesh(core_axis_name="c", subcore_axis_name="s")

    @functools.partial(
        pl.kernel,
        out_type=(
            jax.ShapeDtypeStruct((N_TOK, N_EMBD), jnp.float32),
            jax.ShapeDtypeStruct((N_TOK, N_EMBD), jnp.float32),
        ),
        mesh=mesh,
        scratch_types=[
            pltpu.VMEM((per_w,), jnp.int32),
            pltpu.VMEM((per_w,), jnp.int32),
            pltpu.VMEM((ch, N_EMBD), jnp.float32),
            pltpu.SemaphoreType.DMA,
        ],
    )
    def k(pa_hbm, pb_hbm, table_hbm, ga_hbm, gb_hbm, pa_v, pb_v, rows_v, sem):
        wid = lax.axis_index("s") * 2 + lax.axis_index("c")
        base = wid * per_w
        pltpu.sync_copy(pa_hbm.at[pl.ds(base, per_w)], pa_v)
        pltpu.sync_copy(pb_hbm.at[pl.ds(base, per_w)], pb_v)

        def body(c, carry):
            pltpu.async_copy(
                table_hbm.at[pa_v.at[pl.ds(c * ch, ch)]], rows_v, sem
            ).wait()
            pltpu.sync_copy(rows_v, ga_hbm.at[pl.ds(base + c * ch, ch)])
            pltpu.async_copy(
                table_hbm.at[pb_v.at[pl.ds(c * ch, ch)]], rows_v, sem
            ).wait()
            pltpu.sync_copy(rows_v, gb_hbm.at[pl.ds(base + c * ch, ch)])
            return carry

        lax.fori_loop(0, per_w // ch, body, 0)

    return k(posA, posB, table)


def _cast_body(x_ref, o_ref):
    o_ref[...] = x_ref[...].astype(jnp.bfloat16)


def _cast_bf16(x):
    n = x.shape[0]
    blk = 512
    return pl.pallas_call(
        _cast_body,
        grid=(n // blk,),
        in_specs=[pl.BlockSpec((blk, N_EMBD), lambda i: (i, 0))],
        out_specs=pl.BlockSpec((blk, N_EMBD), lambda i: (i, 0)),
        out_shape=jax.ShapeDtypeStruct((n, N_EMBD), jnp.bfloat16),
    )(x)


def _add_body(a_ref, b_ref, o_ref):
    o_ref[...] = a_ref[...] + b_ref[...]


def _add(a, b):
    n = a.shape[0]
    blk = 512
    return pl.pallas_call(
        _add_body,
        grid=(n // blk,),
        in_specs=[pl.BlockSpec((blk, N_EMBD), lambda i: (i, 0)),
                  pl.BlockSpec((blk, N_EMBD), lambda i: (i, 0))],
        out_specs=pl.BlockSpec((blk, N_EMBD), lambda i: (i, 0)),
        out_shape=jax.ShapeDtypeStruct((n, N_EMBD), jnp.float32),
    )(a, b)


def _mlp_body(be_ref, xb_ref, w1_ref, w2_ref, prb_ref, o_ref):
    f = pl.program_id(0)
    b = pl.program_id(1)
    xb = xb_ref[...]
    w1b = w1_ref[0].astype(jnp.bfloat16)
    h = lax.dot_general(
        xb, w1b,
        dimension_numbers=(((1,), (1,)), ((), ())),
        preferred_element_type=jnp.float32,
    )
    h = h * (1.0 / (1.0 + jnp.exp(-h)))
    h = h * prb_ref[:, 0:1]
    w2b = w2_ref[0].astype(jnp.bfloat16)
    acc = lax.dot_general(
        h.astype(jnp.bfloat16), w2b,
        dimension_numbers=(((1,), (1,)), ((), ())),
        preferred_element_type=jnp.float32,
    )

    @pl.when(f == 0)
    def _():
        o_ref[pl.ds(b * RB, RB), :] = acc

    @pl.when(f != 0)
    def _():
        o_ref[pl.ds(b * RB, RB), :] += acc


def _mlp(xb, W1, W2, prob8, blk_e):
    grid_spec = pltpu.PrefetchScalarGridSpec(
        num_scalar_prefetch=1,
        grid=(NF, NB),
        in_specs=[
            pl.BlockSpec((RB, N_EMBD), lambda f, b, be: (b, 0)),
            pl.BlockSpec((1, FB, N_EMBD), lambda f, b, be: (be[b], f, 0)),
            pl.BlockSpec((1, N_EMBD, FB), lambda f, b, be: (be[b], 0, f)),
            pl.BlockSpec((RB, N_EXPERT), lambda f, b, be: (b, 0)),
        ],
        out_specs=pl.BlockSpec((NP, N_EMBD), lambda f, b, be: (0, 0)),
    )
    return pl.pallas_call(
        _mlp_body,
        grid_spec=grid_spec,
        out_shape=jax.ShapeDtypeStruct((NP, N_EMBD), jnp.float32),
        compiler_params=pltpu.CompilerParams(
            dimension_semantics=("arbitrary", "arbitrary"),
        ),
    )(blk_e, xb, W1, W2, prob8)


@jax.jit
def kernel(x, Wg, W1, W2):
    B, T, C = x.shape
    x_flat = x.reshape(-1, C)
    idx8, prb8 = _gate(x_flat, Wg)
    src, prob, blk_e, posA, posB = _routing_metadata(idx8, prb8)
    xs = _sc_dispatch(src, x_flat)
    xb = _cast_bf16(xs)
    prob8 = jnp.broadcast_to(prob[:, None], (NP, N_EXPERT))
    outs = _mlp(xb, W1, W2, prob8, blk_e)
    gA, gB = _sc_combine(posA, posB, outs)
    y = _add(gA, gB)
    return y.reshape(B, T, C)
